# Initial kernel scaffold; baseline (speedup 1.0000x reference)
import jax
import jax.numpy as jnp
from jax.experimental import pallas as pl


def kernel(entity_table, type_index):
    raise NotImplementedError("write your pallas kernel here")


# speedup vs baseline: 1.8311x; 1.8311x over previous
import functools

import jax
import jax.numpy as jnp
from jax import lax
from jax.experimental import pallas as pl
from jax.experimental.pallas import tpu as pltpu
from jax.experimental.pallas import tpu_sc as plsc

EMBED = 64
NUM_CORES = 2
NUM_SUBCORES = 16
NUM_WORKERS = NUM_CORES * NUM_SUBCORES
CHUNK = 512


def _make_lookup(batch: int):
    b_per_w = batch // NUM_WORKERS
    n_chunks = b_per_w // CHUNK
    mesh = plsc.VectorSubcoreMesh(core_axis_name="c", subcore_axis_name="s")

    @functools.partial(
        pl.kernel,
        mesh=mesh,
        out_type=jax.ShapeDtypeStruct((batch, EMBED), jnp.float32),
        scratch_types=[
            pltpu.VMEM((b_per_w,), jnp.int32),
            pltpu.VMEM((CHUNK, EMBED), jnp.float32),
            pltpu.SemaphoreType.DMA,
        ],
        compiler_params=pltpu.CompilerParams(use_tc_tiling_on_sc=False),
    )
    def lookup(table_hbm, idx_hbm, out_hbm, idx_v, rows_v, sem):
        wid = lax.axis_index("s") * NUM_CORES + lax.axis_index("c")
        base = wid * b_per_w
        pltpu.sync_copy(idx_hbm.at[pl.ds(base, b_per_w)], idx_v)

        def chunk_body(i, carry):
            off = i * CHUNK
            pltpu.async_copy(
                table_hbm.at[idx_v.at[pl.ds(off, CHUNK)]], rows_v, sem
            ).wait()
            pltpu.sync_copy(rows_v, out_hbm.at[pl.ds(base + off, CHUNK)])
            return carry

        lax.fori_loop(0, n_chunks, chunk_body, 0)

    return lookup


def kernel(entity_table, type_index):
    batch, hist = type_index.shape
    idx = type_index.reshape(-1).astype(jnp.int32)
    out = _make_lookup(batch * hist)(entity_table, idx)
    return out.reshape(batch, hist, EMBED)


# speedup vs baseline: 1.8633x; 1.0176x over previous
import functools

import jax
import jax.numpy as jnp
from jax import lax
from jax.experimental import pallas as pl
from jax.experimental.pallas import tpu as pltpu
from jax.experimental.pallas import tpu_sc as plsc

EMBED = 64
NUM_CORES = 2
NUM_SUBCORES = 16
NUM_WORKERS = NUM_CORES * NUM_SUBCORES
CHUNK = 512


def _make_lookup(batch: int):
    b_per_w = batch // NUM_WORKERS
    n_pairs = b_per_w // (2 * CHUNK)
    mesh = plsc.VectorSubcoreMesh(core_axis_name="c", subcore_axis_name="s")

    @functools.partial(
        pl.kernel,
        mesh=mesh,
        out_type=jax.ShapeDtypeStruct((batch, EMBED), jnp.float32),
        scratch_types=[
            pltpu.VMEM((b_per_w,), jnp.int32),
            pltpu.VMEM((2, CHUNK, EMBED), jnp.float32),
            pltpu.SemaphoreType.DMA,
            pltpu.SemaphoreType.DMA,
            pltpu.SemaphoreType.DMA,
            pltpu.SemaphoreType.DMA,
        ],
        compiler_params=pltpu.CompilerParams(use_tc_tiling_on_sc=False),
    )
    def lookup(table_hbm, idx_hbm, out_hbm, idx_v, rows_v, gsem_a, gsem_b,
               osem_a, osem_b):
        wid = lax.axis_index("s") * NUM_CORES + lax.axis_index("c")
        base = wid * b_per_w
        pltpu.sync_copy(idx_hbm.at[pl.ds(base, b_per_w)], idx_v)

        buf_a = rows_v.at[0]
        buf_b = rows_v.at[1]

        def gat(chunk, buf, sem):
            return pltpu.make_async_copy(
                table_hbm.at[idx_v.at[pl.ds(chunk * CHUNK, CHUNK)]], buf, sem)

        def out(chunk, buf, sem):
            return pltpu.make_async_copy(
                buf, out_hbm.at[pl.ds(base + chunk * CHUNK, CHUNK)], sem)

        gat(0, buf_a, gsem_a).start()

        def body(p, carry):
            c0 = 2 * p
            c1 = c0 + 1

            @pl.when(p > 0)
            def _():
                out(c1 - 2, buf_b, osem_b).wait()

            gat(c1, buf_b, gsem_b).start()
            gat(c0, buf_a, gsem_a).wait()
            out(c0, buf_a, osem_a).start()
            gat(c1, buf_b, gsem_b).wait()
            out(c1, buf_b, osem_b).start()

            @pl.when(p < n_pairs - 1)
            def _():
                out(c0, buf_a, osem_a).wait()
                gat(c0 + 2, buf_a, gsem_a).start()

            return carry

        lax.fori_loop(0, n_pairs, body, 0)
        out(2 * n_pairs - 2, buf_a, osem_a).wait()
        out(2 * n_pairs - 1, buf_b, osem_b).wait()

    return lookup


def kernel(entity_table, type_index):
    batch, hist = type_index.shape
    idx = type_index.reshape(-1).astype(jnp.int32)
    out = _make_lookup(batch * hist)(entity_table, idx)
    return out.reshape(batch, hist, EMBED)
